# Initial kernel scaffold; baseline (speedup 1.0000x reference)
import jax
import jax.numpy as jnp
from jax.experimental import pallas as pl


def kernel(diagrams):
    raise NotImplementedError("write your pallas kernel here")


# speedup vs baseline: 7.3294x; 7.3294x over previous
import functools

import jax
import jax.numpy as jnp
import numpy as np
from jax.experimental import pallas as pl
from jax.experimental.pallas import tpu as pltpu
from jax.experimental.pallas import tpu_sc as plsc

K = 128
B = 64
N = 8192
ROW_W = 2 * N
CAP = 512
TRIG = CAP - 64
BUF = CAP + 64
CHUNK_VREGS = 8
NEG = np.float32(-np.inf)


def _vsort(v, desc):
    s, _ = plsc.sort_key_val(v, v, descending=desc)
    return s


def _rev(v):
    return jax.lax.rev(v, (0,))


def _ce(vs, i, j, desc):
    a, b = vs[i], vs[j]
    if desc:
        vs[i], vs[j] = jnp.maximum(a, b), jnp.minimum(a, b)
    else:
        vs[i], vs[j] = jnp.minimum(a, b), jnp.maximum(a, b)


def _merge_blocks(vs, desc):
    vs = list(vs)
    n = len(vs)
    s = n // 2
    while s >= 1:
        for base in range(0, n, 2 * s):
            for i in range(base, base + s):
                _ce(vs, i, i + s, desc)
        s //= 2
    return [_vsort(v, desc) for v in vs]


def _sort128(vs):
    r = [_vsort(vs[i], desc=(i % 2 == 0)) for i in range(8)]
    for p in range(4):
        r[2 * p:2 * p + 2] = _merge_blocks(r[2 * p:2 * p + 2], desc=(p % 2 == 0))
    for q in range(2):
        r[4 * q:4 * q + 4] = _merge_blocks(r[4 * q:4 * q + 4], desc=(q % 2 == 0))
    return _merge_blocks(r, desc=True)


def _merge_top(a, b):
    c = [jnp.maximum(a[j], _rev(b[7 - j])) for j in range(8)]
    return _merge_blocks(c, desc=True)


def _load_block(ref, base, cnt, iota):
    vs = []
    for j in range(8):
        off = base + j * 16
        v = ref[pl.ds(off, 16)]
        m = (off + iota) < cnt
        vs.append(jnp.where(m, v, NEG))
    return vs


def _select_top(cbuf, cnt, iota):
    blocks = [_sort128(_load_block(cbuf, blk * 128, cnt, iota))
              for blk in range(CAP // 128)]
    r = blocks[0]
    for blk in blocks[1:]:
        r = _merge_top(r, blk)
    return r


def _popcnt(mask):
    pc = plsc.all_reduce_population_count(mask)
    return jax.lax.squeeze(jax.lax.slice(pc, (0,), (1,)), (0,))


def _sc_body(x_hbm, out_hbm, row_v, cb, cd, outv, cref, tref):
    wid = jax.lax.axis_index("c") * 16 + jax.lax.axis_index("s")
    iota = jax.lax.iota(jnp.int32, 16)
    even = (iota & 1) == 0
    odd = jnp.logical_not(even)
    bufs = (cb, cd)

    def overflow(ch):
        cbuf = bufs[ch]

        def run():
            r = _select_top(cbuf, cref[ch], iota)
            for j in range(8):
                cbuf[pl.ds(j * 16, 16)] = r[j]
            tref[ch] = jnp.min(r[7])
            cref[ch] = 128

        return run

    def row_body(i, _):
        row = wid * 2 + i
        pltpu.sync_copy(x_hbm.at[row], row_v)
        cref[0] = 0
        cref[1] = 0
        tref[0] = NEG
        tref[1] = NEG

        def chunk(ci, _c):
            pl.when(cref[0] >= TRIG)(overflow(0))
            pl.when(cref[1] >= TRIG)(overflow(1))
            tb = tref[0]
            td = tref[1]
            nb = cref[0]
            nd = cref[1]
            base = ci * (CHUNK_VREGS * 16)
            for j in range(CHUNK_VREGS):
                v = row_v[pl.ds(base + j * 16, 16)]
                mb = (v > tb) & even
                md = (v > td) & odd
                plsc.store_compressed(cb.at[pl.ds(nb, 16)], v, mask=mb)
                plsc.store_compressed(cd.at[pl.ds(nd, 16)], v, mask=md)
                nb = nb + _popcnt(mb)
                nd = nd + _popcnt(md)
            cref[0] = nb
            cref[1] = nd
            return _c

        jax.lax.fori_loop(0, ROW_W // (CHUNK_VREGS * 16), chunk, 0)

        for ch in range(2):
            cbuf = bufs[ch]
            cnt = cref[ch]
            blocks = [_sort128(_load_block(cbuf, 128 + blk * 128, cnt, iota))
                      for blk in range(CAP // 128 - 1)]
            t = blocks[0]
            for blk in blocks[1:]:
                t = _merge_top(t, blk)
            b0 = [cbuf[pl.ds(j * 16, 16)] for j in range(8)]
            r = _merge_top(b0, t)
            for j in range(8):
                outv[pl.ds(j * 16, 16)] = r[j]
            pltpu.sync_copy(outv, out_hbm.at[row, pl.ds(ch * K, K)])
        return _

    jax.lax.fori_loop(0, 2, row_body, 0)


def kernel(diagrams):
    x2d = diagrams.reshape(B, ROW_W)
    mesh = plsc.VectorSubcoreMesh(core_axis_name="c", subcore_axis_name="s")
    k = functools.partial(
        pl.kernel,
        mesh=mesh,
        out_type=jax.ShapeDtypeStruct((B, 2 * K), jnp.float32),
        compiler_params=pltpu.CompilerParams(needs_layout_passes=False),
        scratch_types=[
            pltpu.VMEM((ROW_W,), jnp.float32),
            pltpu.VMEM((BUF,), jnp.float32),
            pltpu.VMEM((BUF,), jnp.float32),
            pltpu.VMEM((K,), jnp.float32),
            pltpu.SMEM((2,), jnp.int32),
            pltpu.SMEM((2,), jnp.float32),
        ],
    )(_sc_body)
    return k(x2d)


# speedup vs baseline: 8.5424x; 1.1655x over previous
import functools

import jax
import jax.numpy as jnp
import numpy as np
from jax.experimental import pallas as pl
from jax.experimental.pallas import tpu as pltpu
from jax.experimental.pallas import tpu_sc as plsc

K = 128
B = 64
N = 8192
ROW_W = 2 * N
CAP = 512
SEG_VREGS = 16
TRIG = CAP - SEG_VREGS * 8
BUF = CAP + 64
NEG = np.float32(-np.inf)


def _vsort(v, desc):
    s, _ = plsc.sort_key_val(v, v, descending=desc)
    return s


def _rev(v):
    return jax.lax.rev(v, (0,))


def _ce(vs, i, j, desc):
    a, b = vs[i], vs[j]
    if desc:
        vs[i], vs[j] = jnp.maximum(a, b), jnp.minimum(a, b)
    else:
        vs[i], vs[j] = jnp.minimum(a, b), jnp.maximum(a, b)


def _merge_blocks(vs, desc):
    vs = list(vs)
    n = len(vs)
    s = n // 2
    while s >= 1:
        for base in range(0, n, 2 * s):
            for i in range(base, base + s):
                _ce(vs, i, i + s, desc)
        s //= 2
    return [_vsort(v, desc) for v in vs]


def _sort128(vs):
    r = [_vsort(vs[i], desc=(i % 2 == 0)) for i in range(8)]
    for p in range(4):
        r[2 * p:2 * p + 2] = _merge_blocks(r[2 * p:2 * p + 2], desc=(p % 2 == 0))
    for q in range(2):
        r[4 * q:4 * q + 4] = _merge_blocks(r[4 * q:4 * q + 4], desc=(q % 2 == 0))
    return _merge_blocks(r, desc=True)


def _merge_top(a, b):
    c = [jnp.maximum(a[j], _rev(b[7 - j])) for j in range(8)]
    return _merge_blocks(c, desc=True)


def _load_block(ref, base, cnt, iota):
    vs = []
    for j in range(8):
        off = base + j * 16
        v = ref[pl.ds(off, 16)]
        m = (off + iota) < cnt
        vs.append(jnp.where(m, v, NEG))
    return vs


def _select_top(cbuf, cnt, iota):
    blocks = [_sort128(_load_block(cbuf, blk * 128, cnt, iota))
              for blk in range(CAP // 128)]
    r = blocks[0]
    for blk in blocks[1:]:
        r = _merge_top(r, blk)
    return r


def _popcnt(mask):
    pc = plsc.all_reduce_population_count(mask)
    return jax.lax.squeeze(jax.lax.slice(pc, (0,), (1,)), (0,))


def _sc_body(x_hbm, out_hbm, rows_v, cb, cd, outv, cref, tref, sem0, sem1):
    wid = jax.lax.axis_index("c") * 16 + jax.lax.axis_index("s")
    iota = jax.lax.iota(jnp.int32, 16)
    even = (iota & 1) == 0
    odd = jnp.logical_not(even)
    bufs = (cb, cd)

    cop0 = pltpu.async_copy(x_hbm.at[wid * 2], rows_v.at[pl.ds(0, ROW_W)], sem0)
    cop1 = pltpu.async_copy(x_hbm.at[wid * 2 + 1], rows_v.at[pl.ds(ROW_W, ROW_W)], sem1)

    def overflow(ch):
        cbuf = bufs[ch]

        def run():
            r = _select_top(cbuf, cref[ch], iota)
            for j in range(8):
                cbuf[pl.ds(j * 16, 16)] = r[j]
            tref[ch] = jnp.min(r[7])
            cref[ch] = 128

        return run

    for i in range(2):
        row = wid * 2 + i
        (cop0 if i == 0 else cop1).wait()
        row_off = i * ROW_W
        cref[0] = 0
        cref[1] = 0
        tref[0] = NEG
        tref[1] = NEG

        def seg_body(si, _c):
            pl.when(cref[0] >= TRIG)(overflow(0))
            pl.when(cref[1] >= TRIG)(overflow(1))
            tb = tref[0]
            td = tref[1]
            base = row_off + si * (SEG_VREGS * 16)

            @plsc.parallel_loop(0, SEG_VREGS, unroll=4,
                                carry=(cref[0], cref[1]))
            def final_cnt(j, c):
                nb, nd = c
                v = rows_v[pl.ds(base + j * 16, 16)]
                mb = (v > tb) & even
                md = (v > td) & odd
                plsc.store_compressed(cb.at[pl.ds(nb, 16)], v, mask=mb)
                plsc.store_compressed(cd.at[pl.ds(nd, 16)], v, mask=md)
                return (nb + _popcnt(mb), nd + _popcnt(md))

            cref[0], cref[1] = final_cnt
            return _c

        jax.lax.fori_loop(0, ROW_W // (SEG_VREGS * 16), seg_body, 0)

        for ch in range(2):
            cbuf = bufs[ch]
            cnt = cref[ch]
            blocks = [_sort128(_load_block(cbuf, 128 + blk * 128, cnt, iota))
                      for blk in range(CAP // 128 - 1)]
            t = blocks[0]
            for blk in blocks[1:]:
                t = _merge_top(t, blk)
            b0 = [cbuf[pl.ds(j * 16, 16)] for j in range(8)]
            r = _merge_top(b0, t)
            for j in range(8):
                outv[pl.ds(j * 16, 16)] = r[j]
            pltpu.sync_copy(outv, out_hbm.at[row, pl.ds(ch * K, K)])


def kernel(diagrams):
    x2d = diagrams.reshape(B, ROW_W)
    mesh = plsc.VectorSubcoreMesh(core_axis_name="c", subcore_axis_name="s")
    k = functools.partial(
        pl.kernel,
        mesh=mesh,
        out_type=jax.ShapeDtypeStruct((B, 2 * K), jnp.float32),
        compiler_params=pltpu.CompilerParams(needs_layout_passes=False),
        scratch_types=[
            pltpu.VMEM((2 * ROW_W,), jnp.float32),
            pltpu.VMEM((BUF,), jnp.float32),
            pltpu.VMEM((BUF,), jnp.float32),
            pltpu.VMEM((K,), jnp.float32),
            pltpu.SMEM((2,), jnp.int32),
            pltpu.SMEM((2,), jnp.float32),
            pltpu.SemaphoreType.DMA,
            pltpu.SemaphoreType.DMA,
        ],
    )(_sc_body)
    return k(x2d)
